# baseline (speedup 1.0000x reference)
import functools
import numpy as np
import jax
from jax import lax
import jax.numpy as jnp
from jax.experimental import pallas as pl
from jax.experimental.pallas import tpu as pltpu
from jax.experimental.pallas import tpu_sc as plsc

N = 10000; D = 128; R = 200; E = 160000; H = 3; B = 1024; L = 2; TOPK = 30; C = 50

NC = 2
NS = 16
NW = NC * NS
NPAD = 10240
ROWS_PER_TILE = NPAD // NS
SCHUNK = 40
VCHUNK = 80


def _seg_kernel_body(vals_hbm, dst_hbm, zer_hbm, sum_hbm,
                     vbuf0, vbuf1, dstv, acc, gsem0, gsem1, ssem0, ssem1):
    cid = lax.axis_index("c")
    sid = lax.axis_index("s")
    wid = cid * NS + sid
    nchunks = dstv.shape[0]
    nouter = nchunks // 5
    ept = nchunks * SCHUNK
    bufs = (vbuf0, vbuf1)
    gsems = (gsem0, gsem1)
    ssems = (ssem0, ssem1)

    pltpu.sync_copy(zer_hbm, acc.at[pl.ds(sid * ROWS_PER_TILE, ROWS_PER_TILE)])
    pltpu.sync_copy(dst_hbm.at[wid], dstv)
    plsc.subcore_barrier()

    def gather(k):
        return pltpu.async_copy(
            vals_hbm.at[pl.ds(wid * ept + k * VCHUNK, VCHUNK)],
            bufs[k % 2], gsems[k % 2])

    pend_scat = {0: [], 1: []}
    g = gather(0)
    for k in range(nouter):
        b = k % 2
        g.wait()
        if k + 1 < nouter:
            for h in pend_scat[1 - b]:
                h.wait()
            pend_scat[1 - b] = []
            g = gather(k + 1)
        for j in range(VCHUNK // SCHUNK):
            idx = dstv.at[k * (VCHUNK // SCHUNK) + j]
            pend_scat[b].append(pltpu.async_copy(
                bufs[b].at[pl.ds(j * SCHUNK, SCHUNK)], acc.at[idx],
                ssems[b], add=True))
    for b in (0, 1):
        for h in pend_scat[b]:
            h.wait()
    plsc.subcore_barrier()

    pltpu.sync_copy(acc.at[pl.ds(sid * ROWS_PER_TILE, ROWS_PER_TILE)],
                    sum_hbm.at[cid, pl.ds(sid * ROWS_PER_TILE, ROWS_PER_TILE)])


def _cnt_kernel_body(dst_hbm, zer_hbm, ones_hbm, cnt_hbm, dstv, onesv, cacc):
    cid = lax.axis_index("c")
    sid = lax.axis_index("s")
    wid = cid * NS + sid
    nchunks = dstv.shape[0]

    pltpu.sync_copy(zer_hbm,
                    cacc.at[pl.ds(sid * ROWS_PER_TILE, ROWS_PER_TILE)])
    pltpu.sync_copy(ones_hbm, onesv)
    pltpu.sync_copy(dst_hbm.at[wid], dstv)
    plsc.subcore_barrier()

    def outer(k, carry):
        pltpu.sync_copy(onesv, cacc.at[dstv.at[k]], add=True)
        return carry

    lax.fori_loop(0, nchunks, outer, 0)
    plsc.subcore_barrier()
    pltpu.sync_copy(cacc.at[pl.ds(sid * ROWS_PER_TILE, ROWS_PER_TILE)],
                    cnt_hbm.at[cid, pl.ds(sid * ROWS_PER_TILE, ROWS_PER_TILE)])


@functools.cache
def _seg_kernel(ep):
    nchunks = ep // NW // SCHUNK
    return pl.kernel(
        _seg_kernel_body,
        mesh=plsc.VectorSubcoreMesh(core_axis_name="c", subcore_axis_name="s"),
        out_type=jax.ShapeDtypeStruct((NC, NPAD, D), jnp.float32),
        scratch_types=[
            pltpu.VMEM((VCHUNK, D), jnp.float32),
            pltpu.VMEM((VCHUNK, D), jnp.float32),
            pltpu.VMEM((nchunks, SCHUNK), jnp.int32),
            pltpu.VMEM_SHARED((NPAD, D), jnp.float32),
            pltpu.SemaphoreType.DMA,
            pltpu.SemaphoreType.DMA,
            pltpu.SemaphoreType.DMA,
            pltpu.SemaphoreType.DMA,
        ],
    )


@functools.cache
def _cnt_kernel(ep):
    nchunks = ep // NW // SCHUNK
    return pl.kernel(
        _cnt_kernel_body,
        mesh=plsc.VectorSubcoreMesh(core_axis_name="c", subcore_axis_name="s"),
        out_type=jax.ShapeDtypeStruct((NC, NPAD, D), jnp.float32),
        scratch_types=[
            pltpu.VMEM((nchunks, SCHUNK), jnp.int32),
            pltpu.VMEM((SCHUNK, D), jnp.float32),
            pltpu.VMEM_SHARED((NPAD, D), jnp.float32),
        ],
    )


def _round_up(x, m):
    return (x + m - 1) // m * m


def _pad_dst(dst):
    e0 = dst.shape[0]
    ep = _round_up(e0, NW * VCHUNK)
    dstp = jnp.concatenate(
        [dst.astype(jnp.int32), jnp.full((ep - e0,), N, jnp.int32)])
    dst3 = dstp.reshape(NW, ep // NW // SCHUNK, SCHUNK)
    return ep, dst3


def _prep_dst(dst):
    ep, dst3 = _pad_dst(dst)
    zer = jnp.zeros((ROWS_PER_TILE, D), jnp.float32)
    ones = jnp.ones((SCHUNK, D), jnp.float32)
    c = _cnt_kernel(ep)(dst3, zer, ones)
    c = (c[0] + c[1])[:N, 0:1]
    return ep, dst3, c


def _prep_dst_nocnt(dst, counts):
    ep, dst3 = _pad_dst(dst)
    return ep, dst3, counts


def _sc_segsum(vals, prep):
    ep, dst3, _ = prep
    vals = jnp.pad(vals, ((0, ep - vals.shape[0]), (0, 0)))
    zer = jnp.zeros((ROWS_PER_TILE, D), jnp.float32)
    s = _seg_kernel(ep)(vals, dst3, zer)
    return (s[0] + s[1])[:N]


def _sc_segmean(vals, prep):
    s = _sc_segsum(vals, prep)
    return s / jnp.maximum(prep[2], 1.0)


def _segsum(v, i, n):
    return jax.ops.segment_sum(v, i, num_segments=n)


def _segmax_c(v, i, n, c):
    m = jax.ops.segment_max(v, i, num_segments=n)
    return jnp.where(c > 0, m, 0.0)


def _segsoftmax(s, i, n):
    e = jnp.exp(s)
    z = _segsum(e, i, n)
    return e / jnp.maximum(z[i], 1e-16)


def _evo(ei, et, rel_emb, hist, tempo_sta, p):
    src, dst = ei[0], ei[1]
    prep = _prep_dst(dst)
    node_mp = _sc_segmean(rel_emb[et], prep)
    g = jax.nn.sigmoid(tempo_sta @ p['gateW_w'] + p['gateW_b'])
    init_ent = g * hist[-1] + (1 - g) * p['nodeemb']
    h = jax.nn.relu(jnp.concatenate([init_ent, node_mp], -1) @ p['sta_W1'] + p['sta_b1'])
    x = h @ p['sta_W2'] + p['sta_b2']
    outs = [x]
    for l in range(L):
        msg = x[src] * rel_emb[et]
        am = _sc_segmean(msg, prep)
        ax = _segmax_c(msg, dst, N, prep[2])
        x = jax.nn.relu(jnp.concatenate([x, am, ax], -1) @ p['lay%d_W' % l] + p['lay%d_b' % l])
        outs.append(x)
    return outs, prep[2]


def _disentangle(curr, last, ei, et, cnt_prev, p):
    an = jnp.arange(N)
    eidx = jnp.concatenate([ei, jnp.stack([an, an])], axis=1)
    temb = jnp.concatenate([p['relemb'][et], jnp.tile(p['selfrel'], (N, 1))], axis=0)
    src, dst = eidx[0], eidx[1]
    q = jnp.concatenate([curr[dst], temb], -1) @ p['q_w'] + p['q_b']
    k = jnp.concatenate([last[src], temb], -1) @ p['k_w'] + p['k_b']
    v = last[src] @ p['v_w'] + p['v_b']
    attn = jnp.sum(q * k, -1) / np.sqrt(D)
    prep = _prep_dst_nocnt(dst, cnt_prev + 1.0)
    e_p = jnp.exp(attn)
    e_m = jnp.exp(-attn)
    zvals = jnp.pad(jnp.stack([e_p, e_m], -1), ((0, 0), (0, D - 2)))
    z = _sc_segsum(zvals, prep)
    fd_num = _sc_segsum(e_p[:, None] * v, prep)
    fs_num = _sc_segsum(e_m[:, None] * v, prep)
    cbar = jnp.maximum(prep[2], 1.0)
    fd = fd_num / (jnp.maximum(z[:, 0:1], 1e-16) * cbar)
    fs = fs_num / (jnp.maximum(z[:, 1:2], 1e-16) * cbar)
    return fs, fd @ p['dyn_weight']


def _gru(x, h, p):
    gi = x @ p['gru_Wi'] + p['gru_bi']
    gh = h @ p['gru_Wh'] + p['gru_bh']
    r = jax.nn.sigmoid(gi[:, :D] + gh[:, :D])
    z = jax.nn.sigmoid(gi[:, D:2 * D] + gh[:, D:2 * D])
    n = jnp.tanh(gi[:, 2 * D:] + r * gh[:, 2 * D:])
    return (1 - z) * n + z * h


def _decoder(node_state, rel_emb, qt2, p):
    e = node_state[qt2[:, 0]]
    r = rel_emb[qt2[:, 1]]
    stk = jnp.stack([e, r], axis=1)
    out = jax.lax.conv_general_dilated(stk, p['dec_conv_w'], (1,), [(1, 1)],
                                       dimension_numbers=('NCH', 'OIH', 'NCH'))
    out = jax.nn.relu(out + p['dec_conv_b'][None, :, None]).reshape(stk.shape[0], C * D)
    x = jax.nn.relu(out @ p['dec_fc_w'] + p['dec_fc_b'])
    return x @ node_state.T


def _forward(ei_all, et_all, qt, p):
    rel_emb = jax.nn.relu(p['relemb'] @ p['W_rel'])
    hist = [p['nodeemb']] * (L + 1)
    td = jnp.zeros((N, D), jnp.float32)
    ts = jnp.zeros((N, D), jnp.float32)
    cnt_prev = jnp.zeros((N, 1), jnp.float32)
    for t in range(H):
        evo, cnt_t = _evo(ei_all[t], et_all[t], rel_emb, hist, ts, p)
        if t == 0:
            lei = jnp.zeros((2, 0), ei_all.dtype)
            let = jnp.zeros((0,), et_all.dtype)
        else:
            lei, let = ei_all[t - 1], et_all[t - 1]
        fs, fd = _disentangle(evo[-1], hist[-1], lei, let, cnt_prev, p)
        cnt_prev = cnt_t
        td = _gru(fd, td, p)
        ts = fs
        hist = evo
    qt2 = qt[:, jnp.array([0, 2, 1])]
    score = _decoder(hist[-1], rel_emb, qt2, p)
    _, tki = jax.lax.top_k(score, TOPK)
    eh = jnp.repeat(qt2[:, 0], TOPK)
    er = jnp.repeat(qt2[:, 1], TOPK)
    etail = tki.reshape(-1)
    vei = jnp.stack([eh, etail])
    virt, _ = _evo(vei, er, rel_emb, hist, ts, p)
    score = _decoder(virt[-1], rel_emb, qt2, p)
    return score


def kernel(edge_index, edge_type, query_triple, relemb, nodeemb, selfrel,
           W_rel, gateW_w, gateW_b, sta_W1, sta_b1, sta_W2, sta_b2,
           lay0_W, lay0_b, lay1_W, lay1_b, q_w, q_b, k_w, k_b, v_w, v_b,
           dyn_weight, gru_Wi, gru_Wh, gru_bi, gru_bh,
           dec_conv_w, dec_conv_b, dec_fc_w, dec_fc_b):
    p = {
        'relemb': relemb, 'nodeemb': nodeemb, 'selfrel': selfrel,
        'W_rel': W_rel, 'gateW_w': gateW_w, 'gateW_b': gateW_b,
        'sta_W1': sta_W1, 'sta_b1': sta_b1, 'sta_W2': sta_W2, 'sta_b2': sta_b2,
        'lay0_W': lay0_W, 'lay0_b': lay0_b, 'lay1_W': lay1_W, 'lay1_b': lay1_b,
        'q_w': q_w, 'q_b': q_b, 'k_w': k_w, 'k_b': k_b, 'v_w': v_w, 'v_b': v_b,
        'dyn_weight': dyn_weight, 'gru_Wi': gru_Wi, 'gru_Wh': gru_Wh,
        'gru_bi': gru_bi, 'gru_bh': gru_bh,
        'dec_conv_w': dec_conv_w, 'dec_conv_b': dec_conv_b,
        'dec_fc_w': dec_fc_w, 'dec_fc_b': dec_fc_b,
    }
    return _forward(edge_index, edge_type, query_triple, p)


# speedup vs baseline: 1.0274x; 1.0274x over previous
import functools
import numpy as np
import jax
from jax import lax
import jax.numpy as jnp
from jax.experimental import pallas as pl
from jax.experimental.pallas import tpu as pltpu
from jax.experimental.pallas import tpu_sc as plsc

N = 10000; D = 128; R = 200; E = 160000; H = 3; B = 1024; L = 2; TOPK = 30; C = 50

NC = 2
NS = 16
NW = NC * NS
NPAD = 10240
ROWS_PER_TILE = NPAD // NS
SCHUNK = 40
VCHUNK = 80


def _seg_kernel_body(vals_hbm, dst_hbm, zer_hbm, sum_hbm,
                     vbuf0, vbuf1, dstv, acc, gsem0, gsem1, ssem0, ssem1):
    cid = lax.axis_index("c")
    sid = lax.axis_index("s")
    wid = cid * NS + sid
    nchunks = dstv.shape[0]
    nouter = nchunks // 5
    ept = nchunks * SCHUNK
    bufs = (vbuf0, vbuf1)
    gsems = (gsem0, gsem1)
    ssems = (ssem0, ssem1)

    pltpu.sync_copy(zer_hbm, acc.at[pl.ds(sid * ROWS_PER_TILE, ROWS_PER_TILE)])
    pltpu.sync_copy(dst_hbm.at[wid], dstv)
    plsc.subcore_barrier()

    def gather(k):
        return pltpu.async_copy(
            vals_hbm.at[pl.ds(wid * ept + k * VCHUNK, VCHUNK)],
            bufs[k % 2], gsems[k % 2])

    pend_scat = {0: [], 1: []}
    g = gather(0)
    for k in range(nouter):
        b = k % 2
        g.wait()
        if k + 1 < nouter:
            for h in pend_scat[1 - b]:
                h.wait()
            pend_scat[1 - b] = []
            g = gather(k + 1)
        for j in range(VCHUNK // SCHUNK):
            idx = dstv.at[k * (VCHUNK // SCHUNK) + j]
            pend_scat[b].append(pltpu.async_copy(
                bufs[b].at[pl.ds(j * SCHUNK, SCHUNK)], acc.at[idx],
                ssems[b], add=True))
    for b in (0, 1):
        for h in pend_scat[b]:
            h.wait()
    plsc.subcore_barrier()

    pltpu.sync_copy(acc.at[pl.ds(sid * ROWS_PER_TILE, ROWS_PER_TILE)],
                    sum_hbm.at[cid, pl.ds(sid * ROWS_PER_TILE, ROWS_PER_TILE)])


def _cnt_kernel_body(dst_hbm, zer_hbm, ones_hbm, cnt_hbm, dstv, onesv, cacc):
    cid = lax.axis_index("c")
    sid = lax.axis_index("s")
    wid = cid * NS + sid
    nchunks = dstv.shape[0]

    pltpu.sync_copy(zer_hbm,
                    cacc.at[pl.ds(sid * ROWS_PER_TILE, ROWS_PER_TILE)])
    pltpu.sync_copy(ones_hbm, onesv)
    pltpu.sync_copy(dst_hbm.at[wid], dstv)
    plsc.subcore_barrier()

    def outer(k, carry):
        pltpu.sync_copy(onesv, cacc.at[dstv.at[k]], add=True)
        return carry

    lax.fori_loop(0, nchunks, outer, 0)
    plsc.subcore_barrier()
    pltpu.sync_copy(cacc.at[pl.ds(sid * ROWS_PER_TILE, ROWS_PER_TILE)],
                    cnt_hbm.at[cid, pl.ds(sid * ROWS_PER_TILE, ROWS_PER_TILE)])


@functools.cache
def _seg_kernel(ep):
    nchunks = ep // NW // SCHUNK
    return pl.kernel(
        _seg_kernel_body,
        mesh=plsc.VectorSubcoreMesh(core_axis_name="c", subcore_axis_name="s"),
        out_type=jax.ShapeDtypeStruct((NC, NPAD, D), jnp.float32),
        scratch_types=[
            pltpu.VMEM((VCHUNK, D), jnp.float32),
            pltpu.VMEM((VCHUNK, D), jnp.float32),
            pltpu.VMEM((nchunks, SCHUNK), jnp.int32),
            pltpu.VMEM_SHARED((NPAD, D), jnp.float32),
            pltpu.SemaphoreType.DMA,
            pltpu.SemaphoreType.DMA,
            pltpu.SemaphoreType.DMA,
            pltpu.SemaphoreType.DMA,
        ],
    )


@functools.cache
def _cnt_kernel(ep):
    nchunks = ep // NW // SCHUNK
    return pl.kernel(
        _cnt_kernel_body,
        mesh=plsc.VectorSubcoreMesh(core_axis_name="c", subcore_axis_name="s"),
        out_type=jax.ShapeDtypeStruct((NC, NPAD, D), jnp.float32),
        scratch_types=[
            pltpu.VMEM((nchunks, SCHUNK), jnp.int32),
            pltpu.VMEM((SCHUNK, D), jnp.float32),
            pltpu.VMEM_SHARED((NPAD, D), jnp.float32),
        ],
    )


def _round_up(x, m):
    return (x + m - 1) // m * m


def _pad_dst(dst):
    e0 = dst.shape[0]
    ep = _round_up(e0, NW * VCHUNK)
    dstp = jnp.concatenate(
        [dst.astype(jnp.int32), jnp.full((ep - e0,), N, jnp.int32)])
    dst3 = dstp.reshape(NW, ep // NW // SCHUNK, SCHUNK)
    return ep, dst3


def _prep_dst(dst):
    ep, dst3 = _pad_dst(dst)
    zer = jnp.zeros((ROWS_PER_TILE, D), jnp.float32)
    ones = jnp.ones((SCHUNK, D), jnp.float32)
    c = _cnt_kernel(ep)(dst3, zer, ones)
    c = (c[0] + c[1])[:N, 0:1]
    return ep, dst3, c


def _prep_dst_nocnt(dst, counts):
    ep, dst3 = _pad_dst(dst)
    return ep, dst3, counts


def _sc_segsum(vals, prep):
    ep, dst3, _ = prep
    vals = jnp.pad(vals, ((0, ep - vals.shape[0]), (0, 0)))
    zer = jnp.zeros((ROWS_PER_TILE, D), jnp.float32)
    s = _seg_kernel(ep)(vals, dst3, zer)
    return (s[0] + s[1])[:N]


def _sc_segmean(vals, prep):
    s = _sc_segsum(vals, prep)
    return s / jnp.maximum(prep[2], 1.0)


def _segsum(v, i, n):
    return jax.ops.segment_sum(v, i, num_segments=n)


def _segmax_c(v, i, n, c):
    m = jax.ops.segment_max(v, i, num_segments=n)
    return jnp.where(c > 0, m, 0.0)


def _segsoftmax(s, i, n):
    e = jnp.exp(s)
    z = _segsum(e, i, n)
    return e / jnp.maximum(z[i], 1e-16)


def _evo(ei, et, rel_emb, hist, tempo_sta, p):
    src, dst = ei[0], ei[1]
    prep = _prep_dst(dst)
    node_mp = _sc_segmean(rel_emb[et], prep)
    g = jax.nn.sigmoid(tempo_sta @ p['gateW_w'] + p['gateW_b'])
    init_ent = g * hist[-1] + (1 - g) * p['nodeemb']
    h = jax.nn.relu(jnp.concatenate([init_ent, node_mp], -1) @ p['sta_W1'] + p['sta_b1'])
    x = h @ p['sta_W2'] + p['sta_b2']
    outs = [x]
    for l in range(L):
        msg = x[src] * rel_emb[et]
        am = _sc_segmean(msg, prep)
        ax = _segmax_c(msg, dst, N, prep[2])
        x = jax.nn.relu(jnp.concatenate([x, am, ax], -1) @ p['lay%d_W' % l] + p['lay%d_b' % l])
        outs.append(x)
    return outs, prep[2]


def _disentangle(curr, last, ei, et, cnt_prev, p):
    an = jnp.arange(N)
    eidx = jnp.concatenate([ei, jnp.stack([an, an])], axis=1)
    temb = jnp.concatenate([p['relemb'][et], jnp.tile(p['selfrel'], (N, 1))], axis=0)
    src, dst = eidx[0], eidx[1]
    q = jnp.concatenate([curr[dst], temb], -1) @ p['q_w'] + p['q_b']
    k = jnp.concatenate([last[src], temb], -1) @ p['k_w'] + p['k_b']
    v = last[src] @ p['v_w'] + p['v_b']
    attn = jnp.sum(q * k, -1) / np.sqrt(D)
    prep = _prep_dst_nocnt(dst, cnt_prev + 1.0)
    e_p = jnp.exp(attn)
    e_m = jnp.exp(-attn)
    zvals = jnp.pad(jnp.stack([e_p, e_m], -1), ((0, 0), (0, D - 2)))
    z = _sc_segsum(zvals, prep)
    fd_num = _sc_segsum(e_p[:, None] * v, prep)
    fs_num = _sc_segsum(e_m[:, None] * v, prep)
    cbar = jnp.maximum(prep[2], 1.0)
    fd = fd_num / (jnp.maximum(z[:, 0:1], 1e-16) * cbar)
    fs = fs_num / (jnp.maximum(z[:, 1:2], 1e-16) * cbar)
    return fs, fd @ p['dyn_weight']


def _gru(x, h, p):
    gi = x @ p['gru_Wi'] + p['gru_bi']
    gh = h @ p['gru_Wh'] + p['gru_bh']
    r = jax.nn.sigmoid(gi[:, :D] + gh[:, :D])
    z = jax.nn.sigmoid(gi[:, D:2 * D] + gh[:, D:2 * D])
    n = jnp.tanh(gi[:, 2 * D:] + r * gh[:, 2 * D:])
    return (1 - z) * n + z * h


def _decoder(node_state, rel_emb, qt2, p):
    e = node_state[qt2[:, 0]]
    r = rel_emb[qt2[:, 1]]
    stk = jnp.stack([e, r], axis=1)
    out = jax.lax.conv_general_dilated(stk, p['dec_conv_w'], (1,), [(1, 1)],
                                       dimension_numbers=('NCH', 'OIH', 'NCH'))
    out = jax.nn.relu(out + p['dec_conv_b'][None, :, None]).reshape(stk.shape[0], C * D)
    x = jax.nn.relu(out @ p['dec_fc_w'] + p['dec_fc_b'])
    return x @ node_state.T


def _forward(ei_all, et_all, qt, p):
    rel_emb = jax.nn.relu(p['relemb'] @ p['W_rel'])
    hist = [p['nodeemb']] * (L + 1)
    td = jnp.zeros((N, D), jnp.float32)
    ts = jnp.zeros((N, D), jnp.float32)
    cnt_prev = jnp.zeros((N, 1), jnp.float32)
    for t in range(H):
        evo, cnt_t = _evo(ei_all[t], et_all[t], rel_emb, hist, ts, p)
        if t == 0:
            v0 = hist[-1] @ p['v_w'] + p['v_b']
            fs, fd = v0, v0 @ p['dyn_weight']
        else:
            fs, fd = _disentangle(evo[-1], hist[-1], ei_all[t - 1],
                                  et_all[t - 1], cnt_prev, p)
        cnt_prev = cnt_t
        td = _gru(fd, td, p)
        ts = fs
        hist = evo
    qt2 = qt[:, jnp.array([0, 2, 1])]
    score = _decoder(hist[-1], rel_emb, qt2, p)
    _, tki = jax.lax.top_k(score, TOPK)
    eh = jnp.repeat(qt2[:, 0], TOPK)
    er = jnp.repeat(qt2[:, 1], TOPK)
    etail = tki.reshape(-1)
    vei = jnp.stack([eh, etail])
    virt, _ = _evo(vei, er, rel_emb, hist, ts, p)
    score = _decoder(virt[-1], rel_emb, qt2, p)
    return score


def kernel(edge_index, edge_type, query_triple, relemb, nodeemb, selfrel,
           W_rel, gateW_w, gateW_b, sta_W1, sta_b1, sta_W2, sta_b2,
           lay0_W, lay0_b, lay1_W, lay1_b, q_w, q_b, k_w, k_b, v_w, v_b,
           dyn_weight, gru_Wi, gru_Wh, gru_bi, gru_bh,
           dec_conv_w, dec_conv_b, dec_fc_w, dec_fc_b):
    p = {
        'relemb': relemb, 'nodeemb': nodeemb, 'selfrel': selfrel,
        'W_rel': W_rel, 'gateW_w': gateW_w, 'gateW_b': gateW_b,
        'sta_W1': sta_W1, 'sta_b1': sta_b1, 'sta_W2': sta_W2, 'sta_b2': sta_b2,
        'lay0_W': lay0_W, 'lay0_b': lay0_b, 'lay1_W': lay1_W, 'lay1_b': lay1_b,
        'q_w': q_w, 'q_b': q_b, 'k_w': k_w, 'k_b': k_b, 'v_w': v_w, 'v_b': v_b,
        'dyn_weight': dyn_weight, 'gru_Wi': gru_Wi, 'gru_Wh': gru_Wh,
        'gru_bi': gru_bi, 'gru_bh': gru_bh,
        'dec_conv_w': dec_conv_w, 'dec_conv_b': dec_conv_b,
        'dec_fc_w': dec_fc_w, 'dec_fc_b': dec_fc_b,
    }
    return _forward(edge_index, edge_type, query_triple, p)
